# Initial kernel scaffold; baseline (speedup 1.0000x reference)
import jax
import jax.numpy as jnp
from jax.experimental import pallas as pl


def kernel(x, wq, wk, wv, wo, bo):
    raise NotImplementedError("write your pallas kernel here")


# speedup vs baseline: 2.2055x; 2.2055x over previous
import functools

import jax
import jax.numpy as jnp
from jax import lax
from jax.experimental import pallas as pl
from jax.experimental.pallas import tpu as pltpu

_NUM_HEADS = 16


def _round_up(x, m):
    return (x + m - 1) // m * m


def _qkv_kernel(x_ref, w_ref, qkv_ref):
    x = x_ref[0].astype(jnp.bfloat16)
    acc = jnp.dot(x, w_ref[...], preferred_element_type=jnp.float32)
    qkv_ref[0] = acc.astype(qkv_ref.dtype)


def _attn_kernel(q_ref, k_ref, v_ref, wo_ref, bo_ref, o_ref,
                 m_ref, l_ref, acc_ref, ctx_ref, *, num_heads, head_dim):
    qi = pl.program_id(1)
    ki = pl.program_id(2)
    tq = q_ref.shape[1]
    tk = k_ref.shape[1]

    @pl.when(ki == 0)
    def _init():
        m_ref[...] = jnp.full_like(m_ref, -jnp.inf)
        l_ref[...] = jnp.zeros_like(l_ref)
        acc_ref[...] = jnp.zeros_like(acc_ref)

    @pl.when(ki <= qi)
    def _compute():
        q = q_ref[0]
        k = k_ref[0]
        v = v_ref[0]

        row = qi * tq + lax.broadcasted_iota(jnp.int32, (tq, tk), 0)
        col = ki * tk + lax.broadcasted_iota(jnp.int32, (tq, tk), 1)
        neg = jnp.where(col > row, -jnp.inf, 0.0).astype(jnp.float32)

        for h in range(num_heads):
            lo = h * head_dim
            q_h = q[:, lo:lo + head_dim]
            k_h = k[:, lo:lo + head_dim]
            v_h = v[:, lo:lo + head_dim]

            s = lax.dot_general(q_h, k_h, (((1,), (1,)), ((), ())),
                                preferred_element_type=jnp.float32)
            s = s + neg

            m_prev = m_ref[h]
            m_new = jnp.maximum(m_prev, jnp.max(s, axis=-1, keepdims=True))
            alpha = jnp.exp(m_prev - m_new)
            p = jnp.exp(s - m_new)
            l_ref[h] = alpha * l_ref[h] + jnp.sum(p, axis=-1, keepdims=True)
            acc_ref[h] = alpha * acc_ref[h] + lax.dot_general(
                p.astype(jnp.bfloat16), v_h, (((1,), (0,)), ((), ())),
                preferred_element_type=jnp.float32)
            m_ref[h] = m_new

    @pl.when(ki == qi)
    def _finalize():
        for h in range(num_heads):
            lo = h * head_dim
            inv_l = pl.reciprocal(l_ref[h], approx=False)
            ctx_ref[:, lo:lo + head_dim] = (acc_ref[h] * inv_l).astype(ctx_ref.dtype)
        out = jnp.dot(ctx_ref[...], wo_ref[...],
                      preferred_element_type=jnp.float32)
        o_ref[0] = (out + bo_ref[...].astype(jnp.float32)).astype(o_ref.dtype)


def kernel(x, wq, wk, wv, wo, bo):
    B, T, d_in = x.shape
    d_out = wq.shape[1]
    num_heads = _NUM_HEADS
    head_dim = d_out // num_heads
    scale = 1.0 / (head_dim ** 0.5)

    wqkv = jnp.concatenate([wq * scale, wk, wv], axis=1).astype(jnp.bfloat16)
    bo2 = bo.reshape(1, d_out)

    tp = min(512, _round_up(T, 8))
    t = min(256, _round_up(T, 8))
    T_pad = _round_up(T, max(tp, t))
    if T_pad != T:
        x = jnp.pad(x, ((0, 0), (0, T_pad - T), (0, 0)))
    n_p = T_pad // tp
    n_t = T_pad // t

    qkv = pl.pallas_call(
        _qkv_kernel,
        out_shape=jax.ShapeDtypeStruct((B, T_pad, 3 * d_out), jnp.bfloat16),
        grid=(B, n_p),
        in_specs=[
            pl.BlockSpec((1, tp, d_in), lambda b, i: (b, i, 0)),
            pl.BlockSpec((d_in, 3 * d_out), lambda b, i: (0, 0)),
        ],
        out_specs=pl.BlockSpec((1, tp, 3 * d_out), lambda b, i: (b, i, 0)),
        compiler_params=pltpu.CompilerParams(
            dimension_semantics=("parallel", "parallel")),
    )(x, wqkv)

    out = pl.pallas_call(
        functools.partial(_attn_kernel, num_heads=num_heads, head_dim=head_dim),
        out_shape=jax.ShapeDtypeStruct((B, T_pad, d_out), x.dtype),
        grid=(B, n_t, n_t),
        in_specs=[
            pl.BlockSpec((1, t, d_out), lambda b, qi, ki: (b, qi, 0)),
            pl.BlockSpec((1, t, d_out), lambda b, qi, ki: (b, ki, 1)),
            pl.BlockSpec((1, t, d_out), lambda b, qi, ki: (b, ki, 2)),
            pl.BlockSpec((d_out, d_out), lambda b, qi, ki: (0, 0)),
            pl.BlockSpec((1, d_out), lambda b, qi, ki: (0, 0)),
        ],
        out_specs=pl.BlockSpec((1, t, d_out), lambda b, qi, ki: (b, qi, 0)),
        scratch_shapes=[
            pltpu.VMEM((num_heads, t, 1), jnp.float32),
            pltpu.VMEM((num_heads, t, 1), jnp.float32),
            pltpu.VMEM((num_heads, t, head_dim), jnp.float32),
            pltpu.VMEM((t, d_out), jnp.bfloat16),
        ],
        compiler_params=pltpu.CompilerParams(
            dimension_semantics=("parallel", "parallel", "arbitrary")),
    )(qkv, qkv, qkv, wo.astype(jnp.bfloat16), bo2)

    if T_pad != T:
        out = out[:, :T, :]
    return out


# speedup vs baseline: 3.4002x; 1.5417x over previous
import functools

import jax
import jax.numpy as jnp
from jax import lax
from jax.experimental import pallas as pl
from jax.experimental.pallas import tpu as pltpu

_NUM_HEADS = 16


def _round_up(x, m):
    return (x + m - 1) // m * m


def _qkv_kernel(x_ref, w_ref, qkv_ref):
    x = x_ref[0].astype(jnp.bfloat16)
    acc = jnp.dot(x, w_ref[...], preferred_element_type=jnp.float32)
    qkv_ref[0] = acc.astype(qkv_ref.dtype)


def _attn_kernel(q_ref, k_ref, v_ref, wo_ref, bo_ref, o_ref,
                 m_ref, l_ref, ctx_ref, *, num_heads, head_dim):
    qi = pl.program_id(1)
    ki = pl.program_id(2)
    tq = q_ref.shape[1]
    tk = k_ref.shape[1]

    @pl.when(ki == 0)
    def _init():
        m_ref[...] = jnp.full_like(m_ref, -jnp.inf)
        l_ref[...] = jnp.zeros_like(l_ref)
        ctx_ref[...] = jnp.zeros_like(ctx_ref)

    @pl.when(ki <= qi)
    def _compute():
        q = q_ref[0]
        k = k_ref[0]
        v = v_ref[0]

        kv = ki * tk + lax.broadcasted_iota(jnp.int32, (tk, tq), 0)
        qq = qi * tq + lax.broadcasted_iota(jnp.int32, (tk, tq), 1)
        neg = jnp.where(kv > qq, -jnp.inf, 0.0).astype(jnp.float32)

        for h in range(num_heads):
            lo = h * head_dim
            q_h = q[:, lo:lo + head_dim]
            k_h = k[:, lo:lo + head_dim]
            v_h = v[:, lo:lo + head_dim]

            st = lax.dot_general(k_h, q_h, (((1,), (1,)), ((), ())),
                                 preferred_element_type=jnp.float32)
            st = st + neg

            m_prev = m_ref[h]
            m_new = jnp.maximum(m_prev, jnp.max(st, axis=0, keepdims=True))
            alpha = jnp.exp(m_prev - m_new)
            p = jnp.exp(st - m_new)
            l_ref[h] = alpha * l_ref[h] + jnp.sum(p, axis=0, keepdims=True)
            pv = lax.dot_general(v_h, p.astype(jnp.bfloat16),
                                 (((0,), (0,)), ((), ())),
                                 preferred_element_type=jnp.float32)
            ctx_ref[lo:lo + head_dim, :] = alpha * ctx_ref[lo:lo + head_dim, :] + pv
            m_ref[h] = m_new

    @pl.when(ki == qi)
    def _finalize():
        pieces = []
        for h in range(num_heads):
            lo = h * head_dim
            inv_l = pl.reciprocal(l_ref[h], approx=False)
            pieces.append(ctx_ref[lo:lo + head_dim, :] * inv_l)
        ctx = jnp.concatenate(pieces, axis=0).astype(jnp.bfloat16)
        out = lax.dot_general(ctx, wo_ref[...], (((0,), (0,)), ((), ())),
                              preferred_element_type=jnp.float32)
        o_ref[0] = (out + bo_ref[...].astype(jnp.float32)).astype(o_ref.dtype)


def kernel(x, wq, wk, wv, wo, bo):
    B, T, d_in = x.shape
    d_out = wq.shape[1]
    num_heads = _NUM_HEADS
    head_dim = d_out // num_heads
    scale = 1.0 / (head_dim ** 0.5)

    wqkv = jnp.concatenate([wq * scale, wk, wv], axis=1).astype(jnp.bfloat16)
    bo2 = bo.reshape(1, d_out)

    tp = min(512, _round_up(T, 8))
    t = min(256, _round_up(T, 8))
    T_pad = _round_up(T, max(tp, t))
    if T_pad != T:
        x = jnp.pad(x, ((0, 0), (0, T_pad - T), (0, 0)))
    n_p = T_pad // tp
    n_t = T_pad // t

    qkv = pl.pallas_call(
        _qkv_kernel,
        out_shape=jax.ShapeDtypeStruct((B, T_pad, 3 * d_out), jnp.bfloat16),
        grid=(B, n_p),
        in_specs=[
            pl.BlockSpec((1, tp, d_in), lambda b, i: (b, i, 0)),
            pl.BlockSpec((d_in, 3 * d_out), lambda b, i: (0, 0)),
        ],
        out_specs=pl.BlockSpec((1, tp, 3 * d_out), lambda b, i: (b, i, 0)),
        compiler_params=pltpu.CompilerParams(
            dimension_semantics=("parallel", "parallel")),
    )(x, wqkv)

    out = pl.pallas_call(
        functools.partial(_attn_kernel, num_heads=num_heads, head_dim=head_dim),
        out_shape=jax.ShapeDtypeStruct((B, T_pad, d_out), x.dtype),
        grid=(B, n_t, n_t),
        in_specs=[
            pl.BlockSpec((1, t, d_out), lambda b, qi, ki: (b, qi, 0)),
            pl.BlockSpec((1, t, d_out), lambda b, qi, ki: (b, ki, 1)),
            pl.BlockSpec((1, t, d_out), lambda b, qi, ki: (b, ki, 2)),
            pl.BlockSpec((d_out, d_out), lambda b, qi, ki: (0, 0)),
            pl.BlockSpec((1, d_out), lambda b, qi, ki: (0, 0)),
        ],
        out_specs=pl.BlockSpec((1, t, d_out), lambda b, qi, ki: (b, qi, 0)),
        scratch_shapes=[
            pltpu.VMEM((num_heads, 1, t), jnp.float32),
            pltpu.VMEM((num_heads, 1, t), jnp.float32),
            pltpu.VMEM((d_out, t), jnp.float32),
        ],
        compiler_params=pltpu.CompilerParams(
            dimension_semantics=("parallel", "parallel", "arbitrary")),
    )(qkv, qkv, qkv, wo.astype(jnp.bfloat16), bo2)

    if T_pad != T:
        out = out[:, :T, :]
    return out


# speedup vs baseline: 7.0179x; 2.0640x over previous
import functools
import numpy as np
import jax
import jax.numpy as jnp
from jax import lax
from jax.experimental import pallas as pl
from jax.experimental.pallas import tpu as pltpu

_NUM_HEADS = 16


def _round_up(x, m):
    return (x + m - 1) // m * m


def _qkvt_kernel(x_ref, wt_ref, qkvt_ref):
    x = x_ref[0].astype(jnp.bfloat16)
    acc = lax.dot_general(wt_ref[...], x, (((1,), (1,)), ((), ())),
                          preferred_element_type=jnp.float32)
    qkvt_ref[0] = acc.astype(qkvt_ref.dtype)


def _attn_kernel(q_ref, k_ref, v_ref, wo_ref, bo_ref, o_ref,
                 st_ref, mt_ref, l_ref, ctx_ref, *, num_heads, head_dim, tk):
    qi = pl.program_id(1)
    tq = q_ref.shape[2]
    T = k_ref.shape[2]
    n_k = T // tk

    qt = q_ref[0]

    neg_diag = jnp.where(
        lax.broadcasted_iota(jnp.int32, (tk, tq), 0)
        > lax.broadcasted_iota(jnp.int32, (tk, tq), 1),
        -jnp.inf, 0.0).astype(jnp.float32)

    mt_ref[...] = jnp.full_like(mt_ref, -jnp.inf)

    def _tile_scores(j, masked):
        for h in range(num_heads):
            lo = h * head_dim
            kt_h = k_ref[0, lo:lo + head_dim, j * tk:(j + 1) * tk]
            st = lax.dot_general(kt_h, qt[lo:lo + head_dim, :],
                                 (((0,), (0,)), ((), ())),
                                 preferred_element_type=jnp.float32)
            if masked:
                st = st + neg_diag
            st_ref[h, j * tk:(j + 1) * tk, :] = st
            mt_ref[j, h:h + 1, :] = jnp.max(st, axis=0, keepdims=True)

    for j in range(n_k):
        if j == 0:
            @pl.when(qi == 0)
            def _():
                _tile_scores(0, True)

            @pl.when(qi > 0)
            def _():
                _tile_scores(0, False)
        else:
            @pl.when(j == qi)
            def _():
                _tile_scores(j, True)

            @pl.when(j < qi)
            def _():
                _tile_scores(j, False)

    m_all = mt_ref[0]
    for j in range(1, n_k):
        m_all = jnp.maximum(m_all, mt_ref[j])

    def _tile_accum(j, first):
        for h in range(num_heads):
            lo = h * head_dim
            p = jnp.exp(st_ref[h, j * tk:(j + 1) * tk, :]
                        - m_all[h:h + 1, :])
            psum = jnp.sum(p, axis=0, keepdims=True)
            vt_h = v_ref[0, lo:lo + head_dim, j * tk:(j + 1) * tk]
            pv = lax.dot_general(vt_h, p.astype(jnp.bfloat16),
                                 (((1,), (0,)), ((), ())),
                                 preferred_element_type=jnp.float32)
            if first:
                l_ref[h:h + 1, :] = psum
                ctx_ref[lo:lo + head_dim, :] = pv
            else:
                l_ref[h:h + 1, :] = l_ref[h:h + 1, :] + psum
                ctx_ref[lo:lo + head_dim, :] = ctx_ref[lo:lo + head_dim, :] + pv

    _tile_accum(0, True)
    for j in range(1, n_k):
        @pl.when(j <= qi)
        def _():
            _tile_accum(j, False)

    pieces = []
    for h in range(num_heads):
        lo = h * head_dim
        inv_l = pl.reciprocal(l_ref[h:h + 1, :], approx=False)
        pieces.append(ctx_ref[lo:lo + head_dim, :] * inv_l)
    ctx = jnp.concatenate(pieces, axis=0).astype(jnp.bfloat16)
    out = lax.dot_general(ctx, wo_ref[...], (((0,), (0,)), ((), ())),
                          preferred_element_type=jnp.float32)
    o_ref[0] = (out + bo_ref[...].astype(jnp.float32)).astype(o_ref.dtype)


def kernel(x, wq, wk, wv, wo, bo):
    B, T, d_in = x.shape
    d_out = wq.shape[1]
    num_heads = _NUM_HEADS
    head_dim = d_out // num_heads
    scale = 1.0 / (head_dim ** 0.5)

    wqkvt = jnp.concatenate([wq * scale, wk, wv], axis=1).T.astype(jnp.bfloat16)
    bo2 = bo.reshape(1, d_out)

    tp = min(512, _round_up(T, 8))
    t = min(256, _round_up(T, 8))
    tk = t
    T_pad = _round_up(T, max(tp, t))
    if T_pad != T:
        x = jnp.pad(x, ((0, 0), (0, T_pad - T), (0, 0)))
    n_p = T_pad // tp
    n_t = T_pad // t

    qkvt = pl.pallas_call(
        _qkvt_kernel,
        out_shape=jax.ShapeDtypeStruct((B, 3 * d_out, T_pad), jnp.bfloat16),
        grid=(B, n_p),
        in_specs=[
            pl.BlockSpec((1, tp, d_in), lambda b, i: (b, i, 0)),
            pl.BlockSpec((3 * d_out, d_in), lambda b, i: (0, 0)),
        ],
        out_specs=pl.BlockSpec((1, 3 * d_out, tp), lambda b, i: (b, 0, i)),
        compiler_params=pltpu.CompilerParams(
            dimension_semantics=("parallel", "parallel")),
    )(x, wqkvt)

    out = pl.pallas_call(
        functools.partial(_attn_kernel, num_heads=num_heads,
                          head_dim=head_dim, tk=tk),
        out_shape=jax.ShapeDtypeStruct((B, T_pad, d_out), x.dtype),
        grid=(B, n_t),
        in_specs=[
            pl.BlockSpec((1, d_out, t), lambda b, qi: (b, 0, qi)),
            pl.BlockSpec((1, d_out, T_pad), lambda b, qi: (b, 1, 0)),
            pl.BlockSpec((1, d_out, T_pad), lambda b, qi: (b, 2, 0)),
            pl.BlockSpec((d_out, d_out), lambda b, qi: (0, 0)),
            pl.BlockSpec((1, d_out), lambda b, qi: (0, 0)),
        ],
        out_specs=pl.BlockSpec((1, t, d_out), lambda b, qi: (b, qi, 0)),
        scratch_shapes=[
            pltpu.VMEM((num_heads, T_pad, t), jnp.float32),
            pltpu.VMEM((T_pad // tk, num_heads, t), jnp.float32),
            pltpu.VMEM((num_heads, t), jnp.float32),
            pltpu.VMEM((d_out, t), jnp.float32),
        ],
        compiler_params=pltpu.CompilerParams(
            dimension_semantics=("parallel", "arbitrary")),
    )(qkvt, qkvt, qkvt, wo.astype(jnp.bfloat16), bo2)

    if T_pad != T:
        out = out[:, :T, :]
    return out
